# Initial kernel scaffold; baseline (speedup 1.0000x reference)
import jax
import jax.numpy as jnp
from jax.experimental import pallas as pl


def kernel(x):
    raise NotImplementedError("write your pallas kernel here")


# speedup vs baseline: 16.1699x; 16.1699x over previous
import functools

import jax
import jax.numpy as jnp
from jax import lax
from jax.experimental import pallas as pl

TOPK_K = 128
ROWS_PER_BLOCK = 256


def _topk_mask_body(x_ref, o_ref):
    xb = x_ref[...]
    b = lax.bitcast_convert_type(xb, jnp.int32)
    ub = lax.bitcast_convert_type(xb, jnp.uint32)
    ukey = jnp.where(b < 0, ~ub, ub | jnp.uint32(0x80000000))

    def bit_step(i, t):
        bit = jnp.uint32(31) - i.astype(jnp.uint32)
        cand = t | (jnp.uint32(1) << bit)
        cnt = jnp.sum((ukey >= cand).astype(jnp.int32), axis=1, keepdims=True)
        return jnp.where(cnt >= TOPK_K, cand, t)

    t0 = jnp.zeros((xb.shape[0], 1), jnp.uint32)
    t = lax.fori_loop(0, 32, bit_step, t0)
    o_ref[...] = jnp.where(ukey >= t, xb, 0.0)


def kernel(x):
    n, c, h, w = x.shape
    rows = n * c
    s = h * w
    xr = x.reshape(rows, s)
    r = ROWS_PER_BLOCK
    out = pl.pallas_call(
        _topk_mask_body,
        grid=(rows // r,),
        in_specs=[pl.BlockSpec((r, s), lambda i: (i, 0))],
        out_specs=pl.BlockSpec((r, s), lambda i: (i, 0)),
        out_shape=jax.ShapeDtypeStruct((rows, s), x.dtype),
    )(xr)
    return out.reshape(n, c, h, w)
